# Initial kernel scaffold; baseline (speedup 1.0000x reference)
import jax
import jax.numpy as jnp
from jax.experimental import pallas as pl


def kernel(x, edge_index, W1, b1, W2, b2, W3, b3, Wc, bc):
    raise NotImplementedError("write your pallas kernel here")


# speedup vs baseline: 23.9893x; 23.9893x over previous
import functools

import jax
import jax.numpy as jnp
from jax import lax
from jax.experimental import pallas as pl
from jax.experimental.pallas import tpu as pltpu
from jax.experimental.pallas import tpu_sc as plsc

NC = 2
NS = 16
NW = NC * NS
CL = 128
FP = 16

_SC_PARAMS = pltpu.CompilerParams(use_tc_tiling_on_sc=False)


def _sc_mesh():
    return plsc.VectorSubcoreMesh(
        core_axis_name="c", subcore_axis_name="s", num_cores=NC, num_subcores=NS
    )


def _make_deg_kernel(n_pad, cw):
    rows_w = n_pad // NS

    @functools.partial(
        pl.kernel,
        out_type=jax.ShapeDtypeStruct((NC, n_pad, FP), jnp.float32),
        mesh=_sc_mesh(),
        compiler_params=_SC_PARAMS,
        scratch_types=[
            pltpu.VMEM((cw, CL), jnp.int32),
            pltpu.VMEM((CL, FP), jnp.float32),
            pltpu.VMEM((rows_w, FP), jnp.float32),
            pltpu.VMEM_SHARED((n_pad, FP), jnp.float32),
        ],
    )
    def deg_kernel(dst_hbm, ones_hbm, zeros_hbm, out_hbm, dst_v, ones_v, buf_v, acc_sh):
        c = lax.axis_index("c")
        s = lax.axis_index("s")
        w = c * NS + s
        pltpu.sync_copy(dst_hbm.at[pl.ds(w * cw, cw)], dst_v)
        pltpu.sync_copy(ones_hbm, ones_v)
        pltpu.sync_copy(zeros_hbm.at[pl.ds(s * rows_w, rows_w)], buf_v)
        pltpu.sync_copy(buf_v, acc_sh.at[pl.ds(s * rows_w, rows_w)])
        plsc.subcore_barrier()

        def body(k, carry):
            pltpu.sync_copy(ones_v, acc_sh.at[dst_v.at[k]], add=True)
            return carry

        lax.fori_loop(0, cw, body, 0)
        plsc.subcore_barrier()
        pltpu.sync_copy(acc_sh.at[pl.ds(s * rows_w, rows_w)], buf_v)
        pltpu.sync_copy(buf_v, out_hbm.at[c, pl.ds(s * rows_w, rows_w)])

    return deg_kernel


def _make_edge_kernel(n_pad, cw):
    rows_w = n_pad // NS

    @functools.partial(
        pl.kernel,
        out_type=jax.ShapeDtypeStruct((NC, n_pad, FP), jnp.float32),
        mesh=_sc_mesh(),
        compiler_params=_SC_PARAMS,
        scratch_types=[
            pltpu.VMEM((cw, CL), jnp.int32),
            pltpu.VMEM((cw, CL), jnp.int32),
            pltpu.VMEM((CL, FP), jnp.float32),
            pltpu.VMEM((rows_w, FP), jnp.float32),
            pltpu.VMEM_SHARED((n_pad, FP), jnp.float32),
            pltpu.SemaphoreType.DMA,
        ],
    )
    def edge_kernel(
        y_hbm, src_hbm, dst_hbm, zeros_hbm, out_hbm,
        src_v, dst_v, rows_v, buf_v, acc_sh, sem,
    ):
        c = lax.axis_index("c")
        s = lax.axis_index("s")
        w = c * NS + s
        pltpu.sync_copy(src_hbm.at[pl.ds(w * cw, cw)], src_v)
        pltpu.sync_copy(dst_hbm.at[pl.ds(w * cw, cw)], dst_v)
        pltpu.sync_copy(zeros_hbm.at[pl.ds(s * rows_w, rows_w)], buf_v)
        pltpu.sync_copy(buf_v, acc_sh.at[pl.ds(s * rows_w, rows_w)])
        plsc.subcore_barrier()

        def body(k, carry):
            pltpu.async_copy(y_hbm.at[src_v.at[k]], rows_v, sem).wait()
            pltpu.sync_copy(rows_v, acc_sh.at[dst_v.at[k]], add=True)
            return carry

        lax.fori_loop(0, cw, body, 0)
        plsc.subcore_barrier()
        pltpu.sync_copy(acc_sh.at[pl.ds(s * rows_w, rows_w)], buf_v)
        pltpu.sync_copy(buf_v, out_hbm.at[c, pl.ds(s * rows_w, rows_w)])

    return edge_kernel


def _tc_prep(x_ref, w_ref, dega_ref, dinv_ref, y_ref):
    deg = dega_ref[0, :, 0:1] + dega_ref[1, :, 0:1] + 1.0
    dinv = lax.rsqrt(deg)
    dinv_ref[...] = dinv
    xw = jnp.dot(x_ref[...], w_ref[...], preferred_element_type=jnp.float32)
    y_ref[...] = dinv * xw


def _tc_layer(acc_ref, y_ref, dinv_ref, b_ref, w_ref, ynext_ref):
    dinv = dinv_ref[...]
    h = jnp.tanh(dinv * (acc_ref[0] + acc_ref[1] + y_ref[...]) + b_ref[...])
    ynext_ref[...] = dinv * jnp.dot(
        h, w_ref[...], preferred_element_type=jnp.float32
    )


def _tc_final(acc_ref, y_ref, dinv_ref, b_ref, wc_ref, bc_ref, out_ref, h_ref):
    dinv = dinv_ref[...]
    h = jnp.tanh(dinv * (acc_ref[0] + acc_ref[1] + y_ref[...]) + b_ref[...])
    h_ref[...] = h[:, 0:2]
    out_ref[...] = (
        jnp.dot(h, wc_ref[...], preferred_element_type=jnp.float32) + bc_ref[...]
    )


def kernel(x, edge_index, W1, b1, W2, b2, W3, b3, Wc, bc):
    n, d_feat = x.shape
    e = edge_index.shape[1]
    f32 = jnp.float32

    n_pad = ((n + 1 + CL - 1) // CL) * CL
    cw = (e + NW * CL - 1) // (NW * CL)
    cw = ((cw + 7) // 8) * 8
    e_pad = NW * cw * CL

    pad = jnp.full((e_pad - e,), n, dtype=jnp.int32)
    src_p = jnp.concatenate([edge_index[0], pad]).reshape(NW * cw, CL)
    dst_p = jnp.concatenate([edge_index[1], pad]).reshape(NW * cw, CL)
    x_pad = jnp.pad(x, ((0, n_pad - n), (0, 0)))

    W1p = jnp.pad(W1, ((0, 0), (0, FP - W1.shape[1])))
    W2p = jnp.pad(W2, ((0, FP - W2.shape[0]), (0, FP - W2.shape[1])))
    W3p = jnp.pad(W3, ((0, FP - W3.shape[0]), (0, FP - W3.shape[1])))
    Wcp = jnp.pad(Wc, ((0, FP - Wc.shape[0]), (0, 0)))
    b1p = jnp.pad(b1, (0, FP - b1.shape[0])).reshape(1, FP)
    b2p = jnp.pad(b2, (0, FP - b2.shape[0])).reshape(1, FP)
    b3p = jnp.pad(b3, (0, FP - b3.shape[0])).reshape(1, FP)
    bcp = bc.reshape(1, -1)

    ones_r = jnp.ones((CL, FP), f32)
    zeros_r = jnp.zeros((n_pad, FP), f32)

    dega = _make_deg_kernel(n_pad, cw)(dst_p, ones_r, zeros_r)

    dinv, y1 = pl.pallas_call(
        _tc_prep,
        out_shape=[
            jax.ShapeDtypeStruct((n_pad, 1), f32),
            jax.ShapeDtypeStruct((n_pad, FP), f32),
        ],
    )(x_pad, W1p, dega)

    edge = _make_edge_kernel(n_pad, cw)

    acc1 = edge(y1, src_p, dst_p, zeros_r)
    y2 = pl.pallas_call(
        _tc_layer, out_shape=jax.ShapeDtypeStruct((n_pad, FP), f32)
    )(acc1, y1, dinv, b1p, W2p)

    acc2 = edge(y2, src_p, dst_p, zeros_r)
    y3 = pl.pallas_call(
        _tc_layer, out_shape=jax.ShapeDtypeStruct((n_pad, FP), f32)
    )(acc2, y2, dinv, b2p, W3p)

    acc3 = edge(y3, src_p, dst_p, zeros_r)
    out, h = pl.pallas_call(
        _tc_final,
        out_shape=[
            jax.ShapeDtypeStruct((n_pad, 10), f32),
            jax.ShapeDtypeStruct((n_pad, 2), f32),
        ],
    )(acc3, y3, dinv, b3p, Wcp, bcp)

    return (out[:n], h[:n])


# speedup vs baseline: 30.9548x; 1.2904x over previous
import functools

import jax
import jax.numpy as jnp
from jax import lax
from jax.experimental import pallas as pl
from jax.experimental.pallas import tpu as pltpu
from jax.experimental.pallas import tpu_sc as plsc

NC = 2
NS = 16
NW = NC * NS
CL = 128
FP = 16

_SC_PARAMS = pltpu.CompilerParams(use_tc_tiling_on_sc=False)


def _sc_mesh():
    return plsc.VectorSubcoreMesh(
        core_axis_name="c", subcore_axis_name="s", num_cores=NC, num_subcores=NS
    )


def _make_deg_kernel(n_pad, cw):
    rows_w = n_pad // NS

    @functools.partial(
        pl.kernel,
        out_type=jax.ShapeDtypeStruct((NC, n_pad, FP), jnp.float32),
        mesh=_sc_mesh(),
        compiler_params=_SC_PARAMS,
        scratch_types=[
            pltpu.VMEM((cw, CL), jnp.int32),
            pltpu.VMEM((CL, FP), jnp.float32),
            pltpu.VMEM((rows_w, FP), jnp.float32),
            pltpu.VMEM_SHARED((n_pad, FP), jnp.float32),
        ],
    )
    def deg_kernel(dst_hbm, ones_hbm, zeros_hbm, out_hbm, dst_v, ones_v, buf_v, acc_sh):
        c = lax.axis_index("c")
        s = lax.axis_index("s")
        w = c * NS + s
        pltpu.sync_copy(dst_hbm.at[pl.ds(w * cw, cw)], dst_v)
        pltpu.sync_copy(ones_hbm, ones_v)
        pltpu.sync_copy(zeros_hbm.at[pl.ds(s * rows_w, rows_w)], buf_v)
        pltpu.sync_copy(buf_v, acc_sh.at[pl.ds(s * rows_w, rows_w)])
        plsc.subcore_barrier()

        def body(k, carry):
            pltpu.sync_copy(ones_v, acc_sh.at[dst_v.at[k]], add=True)
            return carry

        lax.fori_loop(0, cw, body, 0)
        plsc.subcore_barrier()
        pltpu.sync_copy(acc_sh.at[pl.ds(s * rows_w, rows_w)], buf_v)
        pltpu.sync_copy(buf_v, out_hbm.at[c, pl.ds(s * rows_w, rows_w)])

    return deg_kernel


def _make_edge_kernel(n_pad, cw):
    rows_w = n_pad // NS

    @functools.partial(
        pl.kernel,
        out_type=jax.ShapeDtypeStruct((NC, n_pad, FP), jnp.float32),
        mesh=_sc_mesh(),
        compiler_params=_SC_PARAMS,
        scratch_types=[
            pltpu.VMEM((cw, CL), jnp.int32),
            pltpu.VMEM((cw, CL), jnp.int32),
            pltpu.VMEM((CL, FP), jnp.float32),
            pltpu.VMEM((CL, FP), jnp.float32),
            pltpu.VMEM((rows_w, FP), jnp.float32),
            pltpu.VMEM_SHARED((n_pad, FP), jnp.float32),
            pltpu.SemaphoreType.DMA,
        ],
    )
    def edge_kernel(
        y_hbm, src_hbm, dst_hbm, zeros_hbm, out_hbm,
        src_v, dst_v, rows0_v, rows1_v, buf_v, acc_sh, sem,
    ):
        c = lax.axis_index("c")
        s = lax.axis_index("s")
        w = c * NS + s
        pltpu.sync_copy(src_hbm.at[pl.ds(w * cw, cw)], src_v)
        pltpu.sync_copy(dst_hbm.at[pl.ds(w * cw, cw)], dst_v)
        pltpu.sync_copy(zeros_hbm.at[pl.ds(s * rows_w, rows_w)], buf_v)
        pltpu.sync_copy(buf_v, acc_sh.at[pl.ds(s * rows_w, rows_w)])
        plsc.subcore_barrier()

        pltpu.async_copy(y_hbm.at[src_v.at[0]], rows0_v, sem)
        pltpu.async_copy(y_hbm.at[src_v.at[1]], rows1_v, sem)

        def body(i, carry):
            for b, rows_v in enumerate((rows0_v, rows1_v)):
                k = i * 2 + b
                pltpu.make_async_copy(y_hbm.at[src_v.at[k]], rows_v, sem).wait()
                pltpu.sync_copy(rows_v, acc_sh.at[dst_v.at[k]], add=True)

                @pl.when(k + 2 < cw)
                def _():
                    pltpu.async_copy(y_hbm.at[src_v.at[k + 2]], rows_v, sem)

            return carry

        lax.fori_loop(0, cw // 2, body, 0)
        plsc.subcore_barrier()
        pltpu.sync_copy(acc_sh.at[pl.ds(s * rows_w, rows_w)], buf_v)
        pltpu.sync_copy(buf_v, out_hbm.at[c, pl.ds(s * rows_w, rows_w)])

    return edge_kernel


def _tc_prep(x_ref, w_ref, dega_ref, dinv_ref, y_ref):
    deg = dega_ref[0, :, 0:1] + dega_ref[1, :, 0:1] + 1.0
    dinv = lax.rsqrt(deg)
    dinv_ref[...] = dinv
    xw = jnp.dot(x_ref[...], w_ref[...], preferred_element_type=jnp.float32)
    y_ref[...] = dinv * xw


def _tc_layer(acc_ref, y_ref, dinv_ref, b_ref, w_ref, ynext_ref):
    dinv = dinv_ref[...]
    h = jnp.tanh(dinv * (acc_ref[0] + acc_ref[1] + y_ref[...]) + b_ref[...])
    ynext_ref[...] = dinv * jnp.dot(
        h, w_ref[...], preferred_element_type=jnp.float32
    )


def _tc_final(acc_ref, y_ref, dinv_ref, b_ref, wc_ref, bc_ref, out_ref, h_ref):
    dinv = dinv_ref[...]
    h = jnp.tanh(dinv * (acc_ref[0] + acc_ref[1] + y_ref[...]) + b_ref[...])
    h_ref[...] = h[:, 0:2]
    out_ref[...] = (
        jnp.dot(h, wc_ref[...], preferred_element_type=jnp.float32) + bc_ref[...]
    )


def kernel(x, edge_index, W1, b1, W2, b2, W3, b3, Wc, bc):
    n, d_feat = x.shape
    e = edge_index.shape[1]
    f32 = jnp.float32

    n_pad = ((n + 1 + CL - 1) // CL) * CL
    cw = (e + NW * CL - 1) // (NW * CL)
    cw = ((cw + 7) // 8) * 8
    e_pad = NW * cw * CL

    pad = jnp.full((e_pad - e,), n, dtype=jnp.int32)
    src_p = jnp.concatenate([edge_index[0], pad]).reshape(NW * cw, CL)
    dst_p = jnp.concatenate([edge_index[1], pad]).reshape(NW * cw, CL)
    x_pad = jnp.pad(x, ((0, n_pad - n), (0, 0)))

    W1p = jnp.pad(W1, ((0, 0), (0, FP - W1.shape[1])))
    W2p = jnp.pad(W2, ((0, FP - W2.shape[0]), (0, FP - W2.shape[1])))
    W3p = jnp.pad(W3, ((0, FP - W3.shape[0]), (0, FP - W3.shape[1])))
    Wcp = jnp.pad(Wc, ((0, FP - Wc.shape[0]), (0, 0)))
    b1p = jnp.pad(b1, (0, FP - b1.shape[0])).reshape(1, FP)
    b2p = jnp.pad(b2, (0, FP - b2.shape[0])).reshape(1, FP)
    b3p = jnp.pad(b3, (0, FP - b3.shape[0])).reshape(1, FP)
    bcp = bc.reshape(1, -1)

    ones_r = jnp.ones((CL, FP), f32)
    zeros_r = jnp.zeros((n_pad, FP), f32)

    dega = _make_deg_kernel(n_pad, cw)(dst_p, ones_r, zeros_r)

    dinv, y1 = pl.pallas_call(
        _tc_prep,
        out_shape=[
            jax.ShapeDtypeStruct((n_pad, 1), f32),
            jax.ShapeDtypeStruct((n_pad, FP), f32),
        ],
    )(x_pad, W1p, dega)

    edge = _make_edge_kernel(n_pad, cw)

    acc1 = edge(y1, src_p, dst_p, zeros_r)
    y2 = pl.pallas_call(
        _tc_layer, out_shape=jax.ShapeDtypeStruct((n_pad, FP), f32)
    )(acc1, y1, dinv, b1p, W2p)

    acc2 = edge(y2, src_p, dst_p, zeros_r)
    y3 = pl.pallas_call(
        _tc_layer, out_shape=jax.ShapeDtypeStruct((n_pad, FP), f32)
    )(acc2, y2, dinv, b2p, W3p)

    acc3 = edge(y3, src_p, dst_p, zeros_r)
    out, h = pl.pallas_call(
        _tc_final,
        out_shape=[
            jax.ShapeDtypeStruct((n_pad, 10), f32),
            jax.ShapeDtypeStruct((n_pad, 2), f32),
        ],
    )(acc3, y3, dinv, b3p, Wcp, bcp)

    return (out[:n], h[:n])


# speedup vs baseline: 50.7935x; 1.6409x over previous
import functools

import jax
import jax.numpy as jnp
from jax import lax
from jax.experimental import pallas as pl
from jax.experimental.pallas import tpu as pltpu
from jax.experimental.pallas import tpu_sc as plsc

NC = 2
NS = 16
NW = NC * NS
CL = 128
FP = 16

_SC_PARAMS = pltpu.CompilerParams(use_tc_tiling_on_sc=False)


def _sc_mesh():
    return plsc.VectorSubcoreMesh(
        core_axis_name="c", subcore_axis_name="s", num_cores=NC, num_subcores=NS
    )


def _make_deg_kernel(n_pad, cw):
    rows_w = n_pad // NS

    @functools.partial(
        pl.kernel,
        out_type=jax.ShapeDtypeStruct((NC, n_pad, FP), jnp.float32),
        mesh=_sc_mesh(),
        compiler_params=_SC_PARAMS,
        scratch_types=[
            pltpu.VMEM((cw, CL), jnp.int32),
            pltpu.VMEM((CL, FP), jnp.float32),
            pltpu.VMEM((rows_w, FP), jnp.float32),
            pltpu.VMEM_SHARED((n_pad, FP), jnp.float32),
        ],
    )
    def deg_kernel(dst_hbm, ones_hbm, zeros_hbm, out_hbm, dst_v, ones_v, buf_v, acc_sh):
        c = lax.axis_index("c")
        s = lax.axis_index("s")
        w = c * NS + s
        pltpu.sync_copy(dst_hbm.at[pl.ds(w * cw, cw)], dst_v)
        pltpu.sync_copy(ones_hbm, ones_v)
        pltpu.sync_copy(zeros_hbm.at[pl.ds(s * rows_w, rows_w)], buf_v)
        pltpu.sync_copy(buf_v, acc_sh.at[pl.ds(s * rows_w, rows_w)])
        plsc.subcore_barrier()

        def body(k, carry):
            pltpu.sync_copy(ones_v, acc_sh.at[dst_v.at[k]], add=True)
            return carry

        lax.fori_loop(0, cw, body, 0)
        plsc.subcore_barrier()
        pltpu.sync_copy(acc_sh.at[pl.ds(s * rows_w, rows_w)], buf_v)
        pltpu.sync_copy(buf_v, out_hbm.at[c, pl.ds(s * rows_w, rows_w)])

    return deg_kernel


def _make_edge_kernel(n_pad, cw):
    rows_w = n_pad // NS

    @functools.partial(
        pl.kernel,
        out_type=jax.ShapeDtypeStruct((NC, n_pad, FP), jnp.float32),
        mesh=_sc_mesh(),
        compiler_params=_SC_PARAMS,
        scratch_types=[
            pltpu.VMEM((cw, CL), jnp.int32),
            pltpu.VMEM((cw, CL), jnp.int32),
            pltpu.VMEM((CL, FP), jnp.float32),
            pltpu.VMEM((CL, FP), jnp.float32),
            pltpu.VMEM((rows_w, FP), jnp.float32),
            pltpu.VMEM_SHARED((n_pad, FP), jnp.float32),
            pltpu.VMEM_SHARED((n_pad, FP), jnp.float32),
            pltpu.SemaphoreType.DMA,
        ],
    )
    def edge_kernel(
        y_hbm, src_hbm, dst_hbm, zeros_hbm, out_hbm,
        src_v, dst_v, rows0_v, rows1_v, buf_v, acc_sh, y_sh, sem,
    ):
        c = lax.axis_index("c")
        s = lax.axis_index("s")
        w = c * NS + s
        pltpu.sync_copy(src_hbm.at[pl.ds(w * cw, cw)], src_v)
        pltpu.sync_copy(dst_hbm.at[pl.ds(w * cw, cw)], dst_v)
        pltpu.sync_copy(y_hbm.at[pl.ds(s * rows_w, rows_w)], buf_v)
        pltpu.sync_copy(buf_v, y_sh.at[pl.ds(s * rows_w, rows_w)])
        pltpu.sync_copy(zeros_hbm.at[pl.ds(s * rows_w, rows_w)], buf_v)
        pltpu.sync_copy(buf_v, acc_sh.at[pl.ds(s * rows_w, rows_w)])
        plsc.subcore_barrier()

        pltpu.async_copy(y_sh.at[src_v.at[0]], rows0_v, sem)
        pltpu.async_copy(y_sh.at[src_v.at[1]], rows1_v, sem)

        def body(i, carry):
            for b, rows_v in enumerate((rows0_v, rows1_v)):
                k = i * 2 + b
                pltpu.make_async_copy(y_sh.at[src_v.at[k]], rows_v, sem).wait()
                pltpu.sync_copy(rows_v, acc_sh.at[dst_v.at[k]], add=True)

                @pl.when(k + 2 < cw)
                def _():
                    pltpu.async_copy(y_sh.at[src_v.at[k + 2]], rows_v, sem)

            return carry

        lax.fori_loop(0, cw // 2, body, 0)
        plsc.subcore_barrier()
        pltpu.sync_copy(acc_sh.at[pl.ds(s * rows_w, rows_w)], buf_v)
        pltpu.sync_copy(buf_v, out_hbm.at[c, pl.ds(s * rows_w, rows_w)])

    return edge_kernel


def _tc_prep(x_ref, w_ref, dega_ref, dinv_ref, y_ref):
    deg = dega_ref[0, :, 0:1] + dega_ref[1, :, 0:1] + 1.0
    dinv = lax.rsqrt(deg)
    dinv_ref[...] = dinv
    xw = jnp.dot(x_ref[...], w_ref[...], preferred_element_type=jnp.float32)
    y_ref[...] = dinv * xw


def _tc_layer(acc_ref, y_ref, dinv_ref, b_ref, w_ref, ynext_ref):
    dinv = dinv_ref[...]
    h = jnp.tanh(dinv * (acc_ref[0] + acc_ref[1] + y_ref[...]) + b_ref[...])
    ynext_ref[...] = dinv * jnp.dot(
        h, w_ref[...], preferred_element_type=jnp.float32
    )


def _tc_final(acc_ref, y_ref, dinv_ref, b_ref, wc_ref, bc_ref, out_ref, h_ref):
    dinv = dinv_ref[...]
    h = jnp.tanh(dinv * (acc_ref[0] + acc_ref[1] + y_ref[...]) + b_ref[...])
    h_ref[...] = h[:, 0:2]
    out_ref[...] = (
        jnp.dot(h, wc_ref[...], preferred_element_type=jnp.float32) + bc_ref[...]
    )


def kernel(x, edge_index, W1, b1, W2, b2, W3, b3, Wc, bc):
    n, d_feat = x.shape
    e = edge_index.shape[1]
    f32 = jnp.float32

    n_pad = ((n + 1 + CL - 1) // CL) * CL
    cw = (e + NW * CL - 1) // (NW * CL)
    cw = ((cw + 7) // 8) * 8
    e_pad = NW * cw * CL

    pad = jnp.full((e_pad - e,), n, dtype=jnp.int32)
    src_p = jnp.concatenate([edge_index[0], pad]).reshape(NW * cw, CL)
    dst_p = jnp.concatenate([edge_index[1], pad]).reshape(NW * cw, CL)
    x_pad = jnp.pad(x, ((0, n_pad - n), (0, 0)))

    W1p = jnp.pad(W1, ((0, 0), (0, FP - W1.shape[1])))
    W2p = jnp.pad(W2, ((0, FP - W2.shape[0]), (0, FP - W2.shape[1])))
    W3p = jnp.pad(W3, ((0, FP - W3.shape[0]), (0, FP - W3.shape[1])))
    Wcp = jnp.pad(Wc, ((0, FP - Wc.shape[0]), (0, 0)))
    b1p = jnp.pad(b1, (0, FP - b1.shape[0])).reshape(1, FP)
    b2p = jnp.pad(b2, (0, FP - b2.shape[0])).reshape(1, FP)
    b3p = jnp.pad(b3, (0, FP - b3.shape[0])).reshape(1, FP)
    bcp = bc.reshape(1, -1)

    ones_r = jnp.ones((CL, FP), f32)
    zeros_r = jnp.zeros((n_pad, FP), f32)

    dega = _make_deg_kernel(n_pad, cw)(dst_p, ones_r, zeros_r)

    dinv, y1 = pl.pallas_call(
        _tc_prep,
        out_shape=[
            jax.ShapeDtypeStruct((n_pad, 1), f32),
            jax.ShapeDtypeStruct((n_pad, FP), f32),
        ],
    )(x_pad, W1p, dega)

    edge = _make_edge_kernel(n_pad, cw)

    acc1 = edge(y1, src_p, dst_p, zeros_r)
    y2 = pl.pallas_call(
        _tc_layer, out_shape=jax.ShapeDtypeStruct((n_pad, FP), f32)
    )(acc1, y1, dinv, b1p, W2p)

    acc2 = edge(y2, src_p, dst_p, zeros_r)
    y3 = pl.pallas_call(
        _tc_layer, out_shape=jax.ShapeDtypeStruct((n_pad, FP), f32)
    )(acc2, y2, dinv, b2p, W3p)

    acc3 = edge(y3, src_p, dst_p, zeros_r)
    out, h = pl.pallas_call(
        _tc_final,
        out_shape=[
            jax.ShapeDtypeStruct((n_pad, 10), f32),
            jax.ShapeDtypeStruct((n_pad, 2), f32),
        ],
    )(acc3, y3, dinv, b3p, Wcp, bcp)

    return (out[:n], h[:n])


# speedup vs baseline: 71.2084x; 1.4019x over previous
import functools

import jax
import jax.numpy as jnp
from jax import lax
from jax.experimental import pallas as pl
from jax.experimental.pallas import tpu as pltpu
from jax.experimental.pallas import tpu_sc as plsc

NC = 2
NS = 16
NW = NC * NS
CL = 128
FP = 16

_SC_PARAMS = pltpu.CompilerParams(use_tc_tiling_on_sc=False)


def _sc_mesh():
    return plsc.VectorSubcoreMesh(
        core_axis_name="c", subcore_axis_name="s", num_cores=NC, num_subcores=NS
    )


def _make_deg_kernel(n_pad, cw):
    rows_w = n_pad // NS

    @functools.partial(
        pl.kernel,
        out_type=jax.ShapeDtypeStruct((NC, n_pad, FP), jnp.float32),
        mesh=_sc_mesh(),
        compiler_params=_SC_PARAMS,
        scratch_types=[
            pltpu.VMEM((cw, CL), jnp.int32),
            pltpu.VMEM((CL, FP), jnp.float32),
            pltpu.VMEM((rows_w, FP), jnp.float32),
            pltpu.VMEM_SHARED((n_pad, FP), jnp.float32),
        ],
    )
    def deg_kernel(dst_hbm, ones_hbm, zeros_hbm, out_hbm, dst_v, ones_v, buf_v, acc_sh):
        c = lax.axis_index("c")
        s = lax.axis_index("s")
        w = c * NS + s
        pltpu.sync_copy(dst_hbm.at[pl.ds(w * cw, cw)], dst_v)
        pltpu.sync_copy(ones_hbm, ones_v)
        pltpu.sync_copy(zeros_hbm.at[pl.ds(s * rows_w, rows_w)], buf_v)
        pltpu.sync_copy(buf_v, acc_sh.at[pl.ds(s * rows_w, rows_w)])
        plsc.subcore_barrier()

        def body(k, carry):
            pltpu.sync_copy(ones_v, acc_sh.at[dst_v.at[k]], add=True)
            return carry

        lax.fori_loop(0, cw, body, 0)
        plsc.subcore_barrier()
        pltpu.sync_copy(acc_sh.at[pl.ds(s * rows_w, rows_w)], buf_v)
        pltpu.sync_copy(buf_v, out_hbm.at[c, pl.ds(s * rows_w, rows_w)])

    return deg_kernel


def _make_edge_kernel(n_pad, cw):
    rows_w = n_pad // NS

    @functools.partial(
        pl.kernel,
        out_type=jax.ShapeDtypeStruct((NC, n_pad, FP), jnp.float32),
        mesh=_sc_mesh(),
        compiler_params=_SC_PARAMS,
        scratch_types=[
            pltpu.VMEM((cw, CL), jnp.int32),
            pltpu.VMEM((cw, CL), jnp.int32),
            pltpu.VMEM((CL, FP), jnp.float32),
            pltpu.VMEM((CL, FP), jnp.float32),
            pltpu.VMEM((rows_w, FP), jnp.float32),
            pltpu.VMEM_SHARED((n_pad, FP), jnp.float32),
            pltpu.VMEM_SHARED((n_pad, FP), jnp.float32),
            pltpu.SemaphoreType.DMA,
        ],
    )
    def edge_kernel(
        y_hbm, src_hbm, dst_hbm, zeros_hbm, out_hbm,
        src_v, dst_v, rows0_v, rows1_v, buf_v, acc_sh, y_sh, sem,
    ):
        c = lax.axis_index("c")
        s = lax.axis_index("s")
        w = c * NS + s
        pltpu.sync_copy(src_hbm.at[pl.ds(w * cw, cw)], src_v)
        pltpu.sync_copy(dst_hbm.at[pl.ds(w * cw, cw)], dst_v)
        pltpu.sync_copy(y_hbm.at[pl.ds(s * rows_w, rows_w)], buf_v)
        pltpu.sync_copy(buf_v, y_sh.at[pl.ds(s * rows_w, rows_w)])
        pltpu.sync_copy(zeros_hbm.at[pl.ds(s * rows_w, rows_w)], buf_v)
        pltpu.sync_copy(buf_v, acc_sh.at[pl.ds(s * rows_w, rows_w)])
        plsc.subcore_barrier()

        pltpu.async_copy(y_sh.at[src_v.at[0]], rows0_v, sem)
        pltpu.async_copy(y_sh.at[src_v.at[1]], rows1_v, sem)

        def body(i, carry):
            for b, rows_v in enumerate((rows0_v, rows1_v)):
                k = i * 2 + b
                pltpu.make_async_copy(y_sh.at[src_v.at[k]], rows_v, sem).wait()
                pltpu.sync_copy(rows_v, acc_sh.at[dst_v.at[k]], add=True)

                @pl.when(k + 2 < cw)
                def _():
                    pltpu.async_copy(y_sh.at[src_v.at[k + 2]], rows_v, sem)

            return carry

        lax.fori_loop(0, cw // 2, body, 0)
        plsc.subcore_barrier()
        pltpu.sync_copy(acc_sh.at[pl.ds(s * rows_w, rows_w)], buf_v)
        pltpu.sync_copy(buf_v, out_hbm.at[c, pl.ds(s * rows_w, rows_w)])

    return edge_kernel


GP = 8


def _tc_prep(x_ref, w_ref, dega_ref, dinv_ref, y_ref):
    dinv = lax.rsqrt(dega_ref[0] + dega_ref[1] + 1.0)
    dinv_ref[...] = dinv
    xw = jnp.dot(x_ref[...], w_ref[...], preferred_element_type=jnp.float32)
    y_ref[...] = dinv * xw


def _tc_layer(acc_ref, y_ref, dinv_ref, b_ref, w_ref, ynext_ref):
    dinv = dinv_ref[...]
    h = jnp.tanh(dinv * (acc_ref[0] + acc_ref[1] + y_ref[...]) + b_ref[...])
    ynext_ref[...] = dinv * jnp.dot(
        h, w_ref[...], preferred_element_type=jnp.float32
    )


def _tc_final(acc_ref, y_ref, dinv_ref, b_ref, wc_ref, bc_ref, sel_ref,
              out_ref, h_ref):
    dinv = dinv_ref[...]
    h = jnp.tanh(dinv * (acc_ref[0] + acc_ref[1] + y_ref[...]) + b_ref[...])
    h_ref[...] = jnp.dot(h, sel_ref[...], preferred_element_type=jnp.float32)
    out_ref[...] = (
        jnp.dot(h, wc_ref[...], preferred_element_type=jnp.float32) + bc_ref[...]
    )


def kernel(x, edge_index, W1, b1, W2, b2, W3, b3, Wc, bc):
    n, d_feat = x.shape
    e = edge_index.shape[1]
    f32 = jnp.float32

    n_pad = ((n + 1 + CL - 1) // CL) * CL
    cw = (e + NW * CL - 1) // (NW * CL)
    cw = ((cw + 7) // 8) * 8
    e_pad = NW * cw * CL

    pad = jnp.full((e_pad - e,), n, dtype=jnp.int32)
    src_p = jnp.concatenate([edge_index[0], pad]).reshape(NW * cw, CL)
    dst_p = jnp.concatenate([edge_index[1], pad]).reshape(NW * cw, CL)
    x_pad = jnp.pad(x, ((0, n_pad - n), (0, 0)))

    rv = n_pad // GP
    eye8 = jnp.eye(GP, dtype=f32)

    W1b = jnp.kron(eye8, jnp.pad(W1, ((0, 0), (0, FP - W1.shape[1]))))
    W2p = jnp.pad(W2, ((0, FP - W2.shape[0]), (0, FP - W2.shape[1])))
    W3p = jnp.pad(W3, ((0, FP - W3.shape[0]), (0, FP - W3.shape[1])))
    Wcp = jnp.pad(Wc, ((0, FP - Wc.shape[0]), (0, 0)))
    W2b = jnp.kron(eye8, W2p)
    W3b = jnp.kron(eye8, W3p)
    Wcb = jnp.kron(eye8, Wcp)
    selb = jnp.kron(eye8, jnp.eye(FP, 2, dtype=f32))
    b1t = jnp.tile(jnp.pad(b1, (0, FP - b1.shape[0])), GP).reshape(1, GP * FP)
    b2t = jnp.tile(jnp.pad(b2, (0, FP - b2.shape[0])), GP).reshape(1, GP * FP)
    b3t = jnp.tile(jnp.pad(b3, (0, FP - b3.shape[0])), GP).reshape(1, GP * FP)
    bct = jnp.tile(bc, GP).reshape(1, GP * bc.shape[0])

    ones_r = jnp.ones((CL, FP), f32)
    zeros_r = jnp.zeros((n_pad, FP), f32)

    dega = _make_deg_kernel(n_pad, cw)(dst_p, ones_r, zeros_r)
    dega_v = dega.reshape(NC, rv, GP * FP)

    dinv_v, y1_v = pl.pallas_call(
        _tc_prep,
        out_shape=[
            jax.ShapeDtypeStruct((rv, GP * FP), f32),
            jax.ShapeDtypeStruct((rv, GP * FP), f32),
        ],
    )(x_pad.reshape(rv, GP * d_feat), W1b, dega_v)

    edge = _make_edge_kernel(n_pad, cw)

    acc1 = edge(y1_v.reshape(n_pad, FP), src_p, dst_p, zeros_r)
    y2_v = pl.pallas_call(
        _tc_layer, out_shape=jax.ShapeDtypeStruct((rv, GP * FP), f32)
    )(acc1.reshape(NC, rv, GP * FP), y1_v, dinv_v, b1t, W2b)

    acc2 = edge(y2_v.reshape(n_pad, FP), src_p, dst_p, zeros_r)
    y3_v = pl.pallas_call(
        _tc_layer, out_shape=jax.ShapeDtypeStruct((rv, GP * FP), f32)
    )(acc2.reshape(NC, rv, GP * FP), y2_v, dinv_v, b2t, W3b)

    acc3 = edge(y3_v.reshape(n_pad, FP), src_p, dst_p, zeros_r)
    out_v, h_v = pl.pallas_call(
        _tc_final,
        out_shape=[
            jax.ShapeDtypeStruct((rv, GP * 10), f32),
            jax.ShapeDtypeStruct((rv, GP * 2), f32),
        ],
    )(acc3.reshape(NC, rv, GP * FP), y3_v, dinv_v, b3t, Wcb, bct, selb)

    return (out_v.reshape(n_pad, 10)[:n], h_v.reshape(n_pad, 2)[:n])


# speedup vs baseline: 75.8230x; 1.0648x over previous
import functools
from math import gcd as _gcd

import jax
import jax.numpy as jnp
from jax import lax
from jax.experimental import pallas as pl
from jax.experimental.pallas import tpu as pltpu
from jax.experimental.pallas import tpu_sc as plsc

NC = 2
NS = 16
NW = NC * NS
CL = 128
FP = 16

_SC_PARAMS = pltpu.CompilerParams(use_tc_tiling_on_sc=False)


def _sc_mesh():
    return plsc.VectorSubcoreMesh(
        core_axis_name="c", subcore_axis_name="s", num_cores=NC, num_subcores=NS
    )


def _load_index_blocks(ei_hbm, row, idx_v, w, cw, n_chunks, blk):
    for j in range(cw // blk):

        @pl.when(w * cw + (j + 1) * blk <= n_chunks)
        def _():
            pltpu.sync_copy(
                ei_hbm.at[row, pl.ds((w * cw + j * blk) * CL, blk * CL)],
                idx_v.at[pl.ds(j * blk * CL, blk * CL)],
            )


def _zero_rows(buf_v, rows):
    zv = jnp.zeros((FP,), jnp.float32)

    def body(i, carry):
        buf_v[i] = zv
        return carry

    lax.fori_loop(0, rows, body, 0)


def _make_deg_kernel(n_pad, cw, n_chunks, blk):
    rows_w = n_pad // NS

    @functools.partial(
        pl.kernel,
        out_type=jax.ShapeDtypeStruct((NC, n_pad, FP), jnp.float32),
        mesh=_sc_mesh(),
        compiler_params=_SC_PARAMS,
        scratch_types=[
            pltpu.VMEM((cw * CL,), jnp.int32),
            pltpu.VMEM((CL, FP), jnp.float32),
            pltpu.VMEM((rows_w, FP), jnp.float32),
            pltpu.VMEM_SHARED((n_pad, FP), jnp.float32),
        ],
    )
    def deg_kernel(ei_hbm, out_hbm, dst_v, ones_v, buf_v, acc_sh):
        c = lax.axis_index("c")
        s = lax.axis_index("s")
        w = c * NS + s
        _load_index_blocks(ei_hbm, 1, dst_v, w, cw, n_chunks, blk)
        ov = jnp.ones((FP,), jnp.float32)

        def fill(i, carry):
            ones_v[i] = ov
            return carry

        lax.fori_loop(0, CL, fill, 0)
        _zero_rows(buf_v, rows_w)
        pltpu.sync_copy(buf_v, acc_sh.at[pl.ds(s * rows_w, rows_w)])
        plsc.subcore_barrier()

        vc = jnp.maximum(0, jnp.minimum(cw, n_chunks - w * cw))

        def body(k, carry):
            pltpu.sync_copy(ones_v, acc_sh.at[dst_v.at[pl.ds(k * CL, CL)]], add=True)
            return carry

        lax.fori_loop(0, vc, body, 0)
        plsc.subcore_barrier()
        pltpu.sync_copy(acc_sh.at[pl.ds(s * rows_w, rows_w)], buf_v)
        pltpu.sync_copy(buf_v, out_hbm.at[c, pl.ds(s * rows_w, rows_w)])

    return deg_kernel


def _make_edge_kernel(n_pad, cw, n_chunks, blk):
    rows_w = n_pad // NS

    @functools.partial(
        pl.kernel,
        out_type=jax.ShapeDtypeStruct((NC, n_pad, FP), jnp.float32),
        mesh=_sc_mesh(),
        compiler_params=_SC_PARAMS,
        scratch_types=[
            pltpu.VMEM((cw * CL,), jnp.int32),
            pltpu.VMEM((cw * CL,), jnp.int32),
            pltpu.VMEM((CL, FP), jnp.float32),
            pltpu.VMEM((CL, FP), jnp.float32),
            pltpu.VMEM((rows_w, FP), jnp.float32),
            pltpu.VMEM_SHARED((n_pad, FP), jnp.float32),
            pltpu.VMEM_SHARED((n_pad, FP), jnp.float32),
            pltpu.SemaphoreType.DMA,
        ],
    )
    def edge_kernel(
        y_hbm, ei_hbm, out_hbm,
        src_v, dst_v, rows0_v, rows1_v, buf_v, acc_sh, y_sh, sem,
    ):
        c = lax.axis_index("c")
        s = lax.axis_index("s")
        w = c * NS + s
        _load_index_blocks(ei_hbm, 0, src_v, w, cw, n_chunks, blk)
        _load_index_blocks(ei_hbm, 1, dst_v, w, cw, n_chunks, blk)
        pltpu.sync_copy(
            y_hbm.at[pl.ds(s * rows_w, rows_w)],
            y_sh.at[pl.ds(s * rows_w, rows_w)],
        )
        _zero_rows(buf_v, rows_w)
        pltpu.sync_copy(buf_v, acc_sh.at[pl.ds(s * rows_w, rows_w)])
        plsc.subcore_barrier()

        vc = jnp.maximum(0, jnp.minimum(cw, n_chunks - w * cw))

        pltpu.async_copy(y_sh.at[src_v.at[pl.ds(0, CL)]], rows0_v, sem)
        pltpu.async_copy(y_sh.at[src_v.at[pl.ds(CL, CL)]], rows1_v, sem)

        def body(i, carry):
            for b, rows_v in enumerate((rows0_v, rows1_v)):
                k = i * 2 + b
                pltpu.make_async_copy(
                    y_sh.at[src_v.at[pl.ds(k * CL, CL)]], rows_v, sem
                ).wait()
                pltpu.sync_copy(
                    rows_v, acc_sh.at[dst_v.at[pl.ds(k * CL, CL)]], add=True
                )

                @pl.when(k + 2 < vc)
                def _():
                    pltpu.async_copy(
                        y_sh.at[src_v.at[pl.ds((k + 2) * CL, CL)]], rows_v, sem
                    )

            return carry

        lax.fori_loop(0, vc // 2, body, 0)
        plsc.subcore_barrier()
        pltpu.sync_copy(acc_sh.at[pl.ds(s * rows_w, rows_w)], buf_v)
        pltpu.sync_copy(buf_v, out_hbm.at[c, pl.ds(s * rows_w, rows_w)])

    return edge_kernel


GP = 8


def _tc_xw(x_ref, w_ref, xw_ref):
    xw_ref[...] = jnp.dot(
        x_ref[...], w_ref[...], preferred_element_type=jnp.float32
    )


def _tc_dinv(dega_ref, xw_ref, dinv_ref, y_ref):
    dinv = lax.rsqrt(dega_ref[0] + dega_ref[1] + 1.0)
    dinv_ref[...] = dinv
    y_ref[...] = dinv * xw_ref[...]


def _tc_layer(acc_ref, y_ref, dinv_ref, b_ref, w_ref, ynext_ref):
    dinv = dinv_ref[...]
    h = jnp.tanh(dinv * (acc_ref[0] + acc_ref[1] + y_ref[...]) + b_ref[...])
    ynext_ref[...] = dinv * jnp.dot(
        h, w_ref[...], preferred_element_type=jnp.float32
    )


def _tc_final(acc_ref, y_ref, dinv_ref, b_ref, wc_ref, bc_ref, sel_ref,
              out_ref, h_ref):
    dinv = dinv_ref[...]
    h = jnp.tanh(dinv * (acc_ref[0] + acc_ref[1] + y_ref[...]) + b_ref[...])
    h_ref[...] = jnp.dot(h, sel_ref[...], preferred_element_type=jnp.float32)
    out_ref[...] = (
        jnp.dot(h, wc_ref[...], preferred_element_type=jnp.float32) + bc_ref[...]
    )


def kernel(x, edge_index, W1, b1, W2, b2, W3, b3, Wc, bc):
    n, d_feat = x.shape
    e = edge_index.shape[1]
    f32 = jnp.float32

    n_pad = ((n + 1 + CL - 1) // CL) * CL
    n_chunks = e // CL
    cw = (n_chunks + NW - 1) // NW
    cw = ((cw + 7) // 8) * 8
    blk = _gcd(cw, n_chunks)

    x_pad = jnp.pad(x, ((0, n_pad - n), (0, 0)))

    rv = n_pad // GP
    eye8 = jnp.eye(GP, dtype=f32)

    W1b = jnp.kron(eye8, jnp.pad(W1, ((0, 0), (0, FP - W1.shape[1]))))
    W2p = jnp.pad(W2, ((0, FP - W2.shape[0]), (0, FP - W2.shape[1])))
    W3p = jnp.pad(W3, ((0, FP - W3.shape[0]), (0, FP - W3.shape[1])))
    Wcp = jnp.pad(Wc, ((0, FP - Wc.shape[0]), (0, 0)))
    W2b = jnp.kron(eye8, W2p)
    W3b = jnp.kron(eye8, W3p)
    Wcb = jnp.kron(eye8, Wcp)
    selb = jnp.kron(eye8, jnp.eye(FP, 2, dtype=f32))
    b1t = jnp.tile(jnp.pad(b1, (0, FP - b1.shape[0])), GP).reshape(1, GP * FP)
    b2t = jnp.tile(jnp.pad(b2, (0, FP - b2.shape[0])), GP).reshape(1, GP * FP)
    b3t = jnp.tile(jnp.pad(b3, (0, FP - b3.shape[0])), GP).reshape(1, GP * FP)
    bct = jnp.tile(bc, GP).reshape(1, GP * bc.shape[0])

    dega = _make_deg_kernel(n_pad, cw, n_chunks, blk)(edge_index)
    dega_v = dega.reshape(NC, rv, GP * FP)
    xw_v = pl.pallas_call(
        _tc_xw, out_shape=jax.ShapeDtypeStruct((rv, GP * FP), f32)
    )(x_pad.reshape(rv, GP * d_feat), W1b)

    dinv_v, y1_v = pl.pallas_call(
        _tc_dinv,
        out_shape=[
            jax.ShapeDtypeStruct((rv, GP * FP), f32),
            jax.ShapeDtypeStruct((rv, GP * FP), f32),
        ],
    )(dega_v, xw_v)

    edge = _make_edge_kernel(n_pad, cw, n_chunks, blk)

    acc1 = edge(y1_v.reshape(n_pad, FP), edge_index)
    y2_v = pl.pallas_call(
        _tc_layer, out_shape=jax.ShapeDtypeStruct((rv, GP * FP), f32)
    )(acc1.reshape(NC, rv, GP * FP), y1_v, dinv_v, b1t, W2b)

    acc2 = edge(y2_v.reshape(n_pad, FP), edge_index)
    y3_v = pl.pallas_call(
        _tc_layer, out_shape=jax.ShapeDtypeStruct((rv, GP * FP), f32)
    )(acc2.reshape(NC, rv, GP * FP), y2_v, dinv_v, b2t, W3b)

    acc3 = edge(y3_v.reshape(n_pad, FP), edge_index)
    out_v, h_v = pl.pallas_call(
        _tc_final,
        out_shape=[
            jax.ShapeDtypeStruct((rv, GP * 10), f32),
            jax.ShapeDtypeStruct((rv, GP * 2), f32),
        ],
    )(acc3.reshape(NC, rv, GP * FP), y3_v, dinv_v, b3t, Wcb, bct, selb)

    return (out_v.reshape(n_pad, 10)[:n], h_v.reshape(n_pad, 2)[:n])


# speedup vs baseline: 84.3096x; 1.1119x over previous
import functools
from math import gcd as _gcd

import jax
import jax.numpy as jnp
from jax import lax
from jax.experimental import pallas as pl
from jax.experimental.pallas import tpu as pltpu
from jax.experimental.pallas import tpu_sc as plsc

NC = 2
NS = 16
NW = NC * NS
CL = 128
FP = 8

_SC_PARAMS = pltpu.CompilerParams(use_tc_tiling_on_sc=False)


def _sc_mesh():
    return plsc.VectorSubcoreMesh(
        core_axis_name="c", subcore_axis_name="s", num_cores=NC, num_subcores=NS
    )


def _load_index_blocks(ei_hbm, idx_v, w, cw, n_chunks, blk):
    for j in range(cw // blk):

        @pl.when(w * cw + (j + 1) * blk <= n_chunks)
        def _():
            pltpu.sync_copy(
                ei_hbm.at[pl.ds((w * cw + j * blk) * CL, blk * CL)],
                idx_v.at[pl.ds(j * blk * CL, blk * CL)],
            )


def _make_deg_kernel(n_pad, cw, n_chunks, blk):
    rows_w = n_pad // NS

    @functools.partial(
        pl.kernel,
        out_type=jax.ShapeDtypeStruct((NC, n_pad, FP), jnp.float32),
        mesh=_sc_mesh(),
        compiler_params=_SC_PARAMS,
        scratch_types=[
            pltpu.VMEM((cw * CL,), jnp.int32),
            pltpu.VMEM((CL, FP), jnp.float32),
            pltpu.VMEM((rows_w, FP), jnp.float32),
            pltpu.VMEM_SHARED((n_pad, FP), jnp.float32),
        ],
    )
    def deg_kernel(ei_hbm, ones_hbm, zeros_hbm, out_hbm, dst_v, ones_v, buf_v, acc_sh):
        c = lax.axis_index("c")
        s = lax.axis_index("s")
        w = c * NS + s
        _load_index_blocks(ei_hbm, dst_v, w, cw, n_chunks, blk)
        pltpu.sync_copy(ones_hbm, ones_v)
        pltpu.sync_copy(zeros_hbm.at[pl.ds(s * rows_w, rows_w)], buf_v)
        pltpu.sync_copy(buf_v, acc_sh.at[pl.ds(s * rows_w, rows_w)])
        plsc.subcore_barrier()

        vc = jnp.maximum(0, jnp.minimum(cw, n_chunks - w * cw))

        def body(k, carry):
            pltpu.sync_copy(ones_v, acc_sh.at[dst_v.at[pl.ds(k * CL, CL)]], add=True)
            return carry

        lax.fori_loop(0, vc, body, 0)
        plsc.subcore_barrier()
        pltpu.sync_copy(acc_sh.at[pl.ds(s * rows_w, rows_w)], buf_v)
        pltpu.sync_copy(buf_v, out_hbm.at[c, pl.ds(s * rows_w, rows_w)])

    return deg_kernel


def _make_edge_kernel(n_pad, cw, n_chunks, blk):
    rows_w = n_pad // NS

    @functools.partial(
        pl.kernel,
        out_type=jax.ShapeDtypeStruct((NC, n_pad, FP), jnp.float32),
        mesh=_sc_mesh(),
        compiler_params=_SC_PARAMS,
        scratch_types=[
            pltpu.VMEM((cw * CL,), jnp.int32),
            pltpu.VMEM((cw * CL,), jnp.int32),
            pltpu.VMEM((CL, FP), jnp.float32),
            pltpu.VMEM((CL, FP), jnp.float32),
            pltpu.VMEM((rows_w, FP), jnp.float32),
            pltpu.VMEM_SHARED((n_pad, FP), jnp.float32),
            pltpu.VMEM_SHARED((n_pad, FP), jnp.float32),
            pltpu.SemaphoreType.DMA,
        ],
    )
    def edge_kernel(
        y_hbm, src_hbm, dst_hbm, zeros_hbm, out_hbm,
        src_v, dst_v, rows0_v, rows1_v, buf_v, acc_sh, y_sh, sem,
    ):
        c = lax.axis_index("c")
        s = lax.axis_index("s")
        w = c * NS + s
        _load_index_blocks(src_hbm, src_v, w, cw, n_chunks, blk)
        _load_index_blocks(dst_hbm, dst_v, w, cw, n_chunks, blk)
        pltpu.sync_copy(
            y_hbm.at[pl.ds(s * rows_w, rows_w)],
            y_sh.at[pl.ds(s * rows_w, rows_w)],
        )
        pltpu.sync_copy(zeros_hbm.at[pl.ds(s * rows_w, rows_w)], buf_v)
        pltpu.sync_copy(buf_v, acc_sh.at[pl.ds(s * rows_w, rows_w)])
        plsc.subcore_barrier()

        vc = jnp.maximum(0, jnp.minimum(cw, n_chunks - w * cw))

        pltpu.async_copy(y_sh.at[src_v.at[pl.ds(0, CL)]], rows0_v, sem)
        pltpu.async_copy(y_sh.at[src_v.at[pl.ds(CL, CL)]], rows1_v, sem)

        def body(i, carry):
            for b, rows_v in enumerate((rows0_v, rows1_v)):
                k = i * 2 + b
                pltpu.make_async_copy(
                    y_sh.at[src_v.at[pl.ds(k * CL, CL)]], rows_v, sem
                ).wait()
                pltpu.sync_copy(
                    rows_v, acc_sh.at[dst_v.at[pl.ds(k * CL, CL)]], add=True
                )

                @pl.when(k + 2 < vc)
                def _():
                    pltpu.async_copy(
                        y_sh.at[src_v.at[pl.ds((k + 2) * CL, CL)]], rows_v, sem
                    )

            return carry

        lax.fori_loop(0, vc // 2, body, 0)
        plsc.subcore_barrier()
        pltpu.sync_copy(acc_sh.at[pl.ds(s * rows_w, rows_w)], buf_v)
        pltpu.sync_copy(buf_v, out_hbm.at[c, pl.ds(s * rows_w, rows_w)])

    return edge_kernel


GP = 128 // FP


def _tc_xw(x_ref, w_ref, xw_ref):
    xw_ref[...] = jnp.dot(
        x_ref[...], w_ref[...], preferred_element_type=jnp.float32
    )


def _tc_dinv(dega_ref, xw_ref, dinv_ref, y_ref):
    dinv = lax.rsqrt(dega_ref[0] + dega_ref[1] + 1.0)
    dinv_ref[...] = dinv
    y_ref[...] = dinv * xw_ref[...]


def _tc_layer(acc_ref, y_ref, dinv_ref, b_ref, w_ref, ynext_ref):
    dinv = dinv_ref[...]
    h = jnp.tanh(dinv * (acc_ref[0] + acc_ref[1] + y_ref[...]) + b_ref[...])
    ynext_ref[...] = dinv * jnp.dot(
        h, w_ref[...], preferred_element_type=jnp.float32
    )


def _tc_final(acc_ref, y_ref, dinv_ref, b_ref, m_ref, bc_ref, out_ref):
    dinv = dinv_ref[...]
    h = jnp.tanh(dinv * (acc_ref[0] + acc_ref[1] + y_ref[...]) + b_ref[...])
    out_ref[...] = (
        jnp.dot(h, m_ref[...], preferred_element_type=jnp.float32) + bc_ref[...]
    )


def kernel(x, edge_index, W1, b1, W2, b2, W3, b3, Wc, bc):
    n, d_feat = x.shape
    e = edge_index.shape[1]
    f32 = jnp.float32

    n_pad = ((n + 1 + CL - 1) // CL) * CL
    n_chunks = e // CL
    cw = (n_chunks + NW - 1) // NW
    cw = ((cw + 7) // 8) * 8
    blk = _gcd(cw, n_chunks)

    x_pad = jnp.pad(x, ((0, n_pad - n), (0, 0)))

    rv = n_pad // GP
    eyeg = jnp.eye(GP, dtype=f32)

    W1b = jnp.kron(eyeg, jnp.pad(W1, ((0, 0), (0, FP - W1.shape[1]))))
    W2p = jnp.pad(W2, ((0, FP - W2.shape[0]), (0, FP - W2.shape[1])))
    W3p = jnp.pad(W3, ((0, FP - W3.shape[0]), (0, FP - W3.shape[1])))
    Wcp = jnp.pad(Wc, ((0, FP - Wc.shape[0]), (0, 0)))
    W2b = jnp.kron(eyeg, W2p)
    W3b = jnp.kron(eyeg, W3p)
    Mb = jnp.kron(eyeg, jnp.concatenate([Wcp, jnp.eye(FP, 2, dtype=f32)], 1))
    b1t = jnp.tile(jnp.pad(b1, (0, FP - b1.shape[0])), GP).reshape(1, GP * FP)
    b2t = jnp.tile(jnp.pad(b2, (0, FP - b2.shape[0])), GP).reshape(1, GP * FP)
    b3t = jnp.tile(jnp.pad(b3, (0, FP - b3.shape[0])), GP).reshape(1, GP * FP)
    nc_out = bc.shape[0] + 2
    bct = jnp.tile(jnp.pad(bc, (0, 2)), GP).reshape(1, GP * nc_out)

    ones_r = jnp.ones((CL, FP), f32)
    zeros_r = jnp.zeros((n_pad, FP), f32)
    src_e = edge_index[0]
    dst_e = edge_index[1]

    dega = _make_deg_kernel(n_pad, cw, n_chunks, blk)(dst_e, ones_r, zeros_r)
    dega_v = dega.reshape(NC, rv, GP * FP)
    xw_v = pl.pallas_call(
        _tc_xw, out_shape=jax.ShapeDtypeStruct((rv, GP * FP), f32)
    )(x_pad.reshape(rv, GP * d_feat), W1b)

    dinv_v, y1_v = pl.pallas_call(
        _tc_dinv,
        out_shape=[
            jax.ShapeDtypeStruct((rv, GP * FP), f32),
            jax.ShapeDtypeStruct((rv, GP * FP), f32),
        ],
    )(dega_v, xw_v)

    edge = _make_edge_kernel(n_pad, cw, n_chunks, blk)

    acc1 = edge(y1_v.reshape(n_pad, FP), src_e, dst_e, zeros_r)
    y2_v = pl.pallas_call(
        _tc_layer, out_shape=jax.ShapeDtypeStruct((rv, GP * FP), f32)
    )(acc1.reshape(NC, rv, GP * FP), y1_v, dinv_v, b1t, W2b)

    acc2 = edge(y2_v.reshape(n_pad, FP), src_e, dst_e, zeros_r)
    y3_v = pl.pallas_call(
        _tc_layer, out_shape=jax.ShapeDtypeStruct((rv, GP * FP), f32)
    )(acc2.reshape(NC, rv, GP * FP), y2_v, dinv_v, b2t, W3b)

    acc3 = edge(y3_v.reshape(n_pad, FP), src_e, dst_e, zeros_r)
    z_v = pl.pallas_call(
        _tc_final, out_shape=jax.ShapeDtypeStruct((rv, GP * nc_out), f32)
    )(acc3.reshape(NC, rv, GP * FP), y3_v, dinv_v, b3t, Mb, bct)

    z = z_v.reshape(n_pad, nc_out)
    return (z[:n, : bc.shape[0]], z[:n, bc.shape[0] :])
